# Initial kernel scaffold; baseline (speedup 1.0000x reference)
import jax
import jax.numpy as jnp
from jax.experimental import pallas as pl


def kernel(relative_pos, edge_index_i, W, b, gamma, beta):
    raise NotImplementedError("write your pallas kernel here")


# speedup vs baseline: 2.4245x; 2.4245x over previous
import functools

import jax
import jax.numpy as jnp
from jax import lax
from jax.experimental import pallas as pl
from jax.experimental.pallas import tpu as pltpu
from jax.experimental.pallas import tpu_sc as plsc

N_NODES = 50000
E_TOTAL = 1600000
C_OUT = 64

NW = 32
E_PER_W = E_TOTAL // NW
CH = 2000
N_CH = E_PER_W // CH

BT = 4000
GRID_E = E_TOTAL // BT
NT = 5000
GRID_N = N_NODES // NT

_INV_SQRT3 = 0.5773502691896258

_sc_mesh = plsc.VectorSubcoreMesh(core_axis_name="c", subcore_axis_name="s")


def _dirs_body(pos_ref, out_ref):
    p = pos_ref[...]
    n = jnp.sqrt(jnp.sum(p * p, axis=1, keepdims=True)) + 1e-8
    d = p / n
    out_ref[...] = jnp.concatenate(
        [d, jnp.ones((BT, 1), jnp.float32),
         jnp.zeros((BT, 4), jnp.float32)], axis=1)


def _dirs(relative_pos):
    return pl.pallas_call(
        _dirs_body,
        grid=(GRID_E,),
        in_specs=[pl.BlockSpec((BT, 3), lambda i: (i, 0))],
        out_specs=pl.BlockSpec((BT, 8), lambda i: (i, 0)),
        out_shape=jax.ShapeDtypeStruct((E_TOTAL, 8), jnp.float32),
    )(relative_pos)


@functools.partial(
    pl.kernel,
    out_type=jax.ShapeDtypeStruct((2, N_NODES, 8), jnp.float32),
    mesh=_sc_mesh,
    compiler_params=pltpu.CompilerParams(use_tc_tiling_on_sc=False),
    scratch_types=[
        pltpu.VMEM((CH,), jnp.int32),
        pltpu.VMEM((CH, 8), jnp.float32),
        pltpu.VMEM_SHARED((N_NODES, 8), jnp.float32),
    ],
)
def _scatter_k(dirs_hbm, idx_hbm, zeros_hbm, part_hbm, idx_v, rows_v, acc_sh):
    c = lax.axis_index("c")
    s = lax.axis_index("s")
    wid = s * 2 + c

    @pl.when(s == 0)
    def _():
        pltpu.sync_copy(zeros_hbm, acc_sh)

    plsc.subcore_barrier()

    def body(i, carry):
        off = wid * E_PER_W + i * CH
        pltpu.sync_copy(idx_hbm.at[pl.ds(off, CH)], idx_v)
        pltpu.sync_copy(dirs_hbm.at[pl.ds(off, CH)], rows_v)
        pltpu.sync_copy(rows_v, acc_sh.at[idx_v], add=True)
        return carry

    lax.fori_loop(0, N_CH, body, 0)
    plsc.subcore_barrier()

    @pl.when(s == 0)
    def _():
        pltpu.sync_copy(acc_sh, part_hbm.at[c])


def _combine_body(p0_ref, p1_ref, out_ref):
    out_ref[...] = p0_ref[...] + p1_ref[...]


def _combine(p0, p1):
    return pl.pallas_call(
        _combine_body,
        grid=(GRID_N,),
        in_specs=[pl.BlockSpec((NT, 8), lambda i: (i, 0)),
                  pl.BlockSpec((NT, 8), lambda i: (i, 0))],
        out_specs=pl.BlockSpec((NT, 8), lambda i: (i, 0)),
        out_shape=jax.ShapeDtypeStruct((N_NODES, 8), jnp.float32),
    )(p0, p1)


@functools.partial(
    pl.kernel,
    out_type=jax.ShapeDtypeStruct((E_TOTAL, 8), jnp.float32),
    mesh=_sc_mesh,
    compiler_params=pltpu.CompilerParams(use_tc_tiling_on_sc=False),
    scratch_types=[
        pltpu.VMEM((CH,), jnp.int32),
        pltpu.VMEM((CH, 8), jnp.float32),
        pltpu.SemaphoreType.DMA,
    ],
)
def _gather_k(table_hbm, idx_hbm, out_hbm, idx_v, rows_v, sem):
    c = lax.axis_index("c")
    s = lax.axis_index("s")
    wid = s * 2 + c

    def body(i, carry):
        off = wid * E_PER_W + i * CH
        pltpu.sync_copy(idx_hbm.at[pl.ds(off, CH)], idx_v)
        pltpu.async_copy(table_hbm.at[idx_v], rows_v, sem).wait()
        pltpu.sync_copy(rows_v, out_hbm.at[pl.ds(off, CH)])
        return carry

    lax.fori_loop(0, N_CH, body, 0)


def _final_body(g_ref, w_ref, b_ref, gam_ref, bet_ref, out_ref):
    g = g_ref[...]
    w = w_ref[...]
    w0, w1, w2, w3 = w[0:1], w[1:2], w[2:3], w[3:4]
    w4, w5, w6, w7 = w[4:5], w[5:6], w[6:7], w[7:8]
    mx = (w0 - w1 + w2 + w3 - w4 - w5 + w6 - w7) * _INV_SQRT3
    my = (w0 + w1 - w2 + w3 - w4 + w5 - w6 - w7) * _INV_SQRT3
    mz = (w0 + w1 + w2 - w3 + w4 - w5 - w6 - w7) * _INV_SQRT3
    inv = 1.0 / jnp.maximum(g[:, 3:4], 1.0)
    feat = ((g[:, 0:1] * inv) * mx + (g[:, 1:2] * inv) * my
            + (g[:, 2:3] * inv) * mz + b_ref[...])
    mu = jnp.mean(feat, axis=1, keepdims=True)
    d = feat - mu
    var = jnp.mean(d * d, axis=1, keepdims=True)
    out_ref[...] = d * lax.rsqrt(var + 1e-5) * gam_ref[...] + bet_ref[...]


def _final(g, W, b, gamma, beta):
    return pl.pallas_call(
        _final_body,
        grid=(GRID_E,),
        in_specs=[
            pl.BlockSpec((BT, 8), lambda i: (i, 0)),
            pl.BlockSpec((8, C_OUT), lambda i: (0, 0)),
            pl.BlockSpec((1, C_OUT), lambda i: (0, 0)),
            pl.BlockSpec((1, C_OUT), lambda i: (0, 0)),
            pl.BlockSpec((1, C_OUT), lambda i: (0, 0)),
        ],
        out_specs=pl.BlockSpec((BT, C_OUT), lambda i: (i, 0)),
        out_shape=jax.ShapeDtypeStruct((E_TOTAL, C_OUT), jnp.float32),
    )(g, W, b.reshape(1, C_OUT), gamma.reshape(1, C_OUT),
      beta.reshape(1, C_OUT))


def kernel(relative_pos, edge_index_i, W, b, gamma, beta):
    dirs4 = _dirs(relative_pos)
    zeros = jnp.zeros((N_NODES, 8), jnp.float32)
    parts = _scatter_k(dirs4, edge_index_i, zeros)
    table = _combine(parts[0], parts[1])
    g = _gather_k(table, edge_index_i)
    return _final(g, W, b, gamma, beta)


# speedup vs baseline: 6.6679x; 2.7502x over previous
import functools

import jax
import jax.numpy as jnp
from jax import lax
from jax.experimental import pallas as pl
from jax.experimental.pallas import tpu as pltpu
from jax.experimental.pallas import tpu_sc as plsc

N_NODES = 50000
E_TOTAL = 1600000
C_OUT = 64

NW = 32
E_PER_W = E_TOTAL // NW
CH = 2000
N_CH = E_PER_W // CH
G16 = CH // 16

TROWS = 3200

BT = 6400
GRID_E = E_TOTAL // BT

_INV_SQRT3 = 0.5773502691896258

_sc_mesh = plsc.VectorSubcoreMesh(core_axis_name="c", subcore_axis_name="s")


def _dirs_body(pos_ref, out_ref):
    p = pos_ref[...]
    x, y, z = p[0:1], p[1:2], p[2:3]
    n = jnp.sqrt(x * x + y * y + z * z) + 1e-8
    out_ref[...] = jnp.concatenate([x / n, y / n, z / n], axis=0)


def _dirs(rp_t):
    return pl.pallas_call(
        _dirs_body,
        grid=(GRID_E,),
        in_specs=[pl.BlockSpec((3, BT), lambda i: (0, i))],
        out_specs=pl.BlockSpec((3, BT), lambda i: (0, i)),
        out_shape=jax.ShapeDtypeStruct((3, E_TOTAL), jnp.float32),
    )(rp_t)


@functools.partial(
    pl.kernel,
    out_type=jax.ShapeDtypeStruct((2, N_NODES, 8), jnp.float32),
    mesh=_sc_mesh,
    compiler_params=pltpu.CompilerParams(use_tc_tiling_on_sc=False, needs_layout_passes=False),
    scratch_types=[
        pltpu.VMEM((CH,), jnp.int32),
        pltpu.VMEM((CH,), jnp.float32),
        pltpu.VMEM((CH,), jnp.float32),
        pltpu.VMEM((CH,), jnp.float32),
        pltpu.VMEM((CH, 8), jnp.float32),
        pltpu.VMEM_SHARED((N_NODES, 8), jnp.float32),
    ],
)
def _scatter_k(dirs_hbm, idx_hbm, zeros_hbm, part_hbm,
               idx_v, cx_v, cy_v, cz_v, rows_v, acc_sh):
    c = lax.axis_index("c")
    s = lax.axis_index("s")
    wid = s * 2 + c

    @pl.when(s == 0)
    def _():
        pltpu.sync_copy(zeros_hbm, acc_sh)

    iota16 = lax.iota(jnp.int32, 16)
    col0 = jnp.zeros((16,), jnp.int32)
    col1 = col0 + 1
    col2 = col0 + 2
    col3 = col0 + 3
    ones16 = jnp.ones((16,), jnp.float32)

    plsc.subcore_barrier()

    def chunk(i, carry):
        off = wid * E_PER_W + i * CH
        pltpu.sync_copy(idx_hbm.at[pl.ds(off, CH)], idx_v)
        pltpu.sync_copy(dirs_hbm.at[0, pl.ds(off, CH)], cx_v)
        pltpu.sync_copy(dirs_hbm.at[1, pl.ds(off, CH)], cy_v)
        pltpu.sync_copy(dirs_hbm.at[2, pl.ds(off, CH)], cz_v)

        def group(j, carry2):
            e16 = iota16 + j * 16
            plsc.store_scatter(rows_v, [e16, col0], cx_v[pl.ds(j * 16, 16)])
            plsc.store_scatter(rows_v, [e16, col1], cy_v[pl.ds(j * 16, 16)])
            plsc.store_scatter(rows_v, [e16, col2], cz_v[pl.ds(j * 16, 16)])
            plsc.store_scatter(rows_v, [e16, col3], ones16)
            return carry2

        lax.fori_loop(0, G16, group, 0)
        pltpu.sync_copy(rows_v, acc_sh.at[idx_v], add=True)
        return carry

    lax.fori_loop(0, N_CH, chunk, 0)
    plsc.subcore_barrier()

    @pl.when(s == 0)
    def _():
        pltpu.sync_copy(acc_sh, part_hbm.at[c])


@functools.partial(
    pl.kernel,
    out_type=jax.ShapeDtypeStruct((4, E_TOTAL), jnp.float32),
    mesh=_sc_mesh,
    compiler_params=pltpu.CompilerParams(use_tc_tiling_on_sc=False, needs_layout_passes=False),
    scratch_types=[
        pltpu.VMEM((CH,), jnp.int32),
        pltpu.VMEM((CH, 8), jnp.float32),
        pltpu.VMEM((CH,), jnp.float32),
        pltpu.VMEM((CH,), jnp.float32),
        pltpu.VMEM((CH,), jnp.float32),
        pltpu.VMEM((CH,), jnp.float32),
        pltpu.VMEM((TROWS, 8), jnp.float32),
        pltpu.VMEM((TROWS,), jnp.int32),
        pltpu.VMEM_SHARED((N_NODES, 8), jnp.float32),
        pltpu.SemaphoreType.DMA,
    ],
)
def _gather_k(part_hbm, idx_hbm, ramp_hbm, out_hbm,
              idx_v, rows_v, gx_v, gy_v, gz_v, gc_v, tmp_v, ramp_v,
              acc_sh, sem):
    c = lax.axis_index("c")
    s = lax.axis_index("s")
    wid = s * 2 + c

    rows0 = s * TROWS

    def build(nr):
        pltpu.sync_copy(part_hbm.at[0, pl.ds(rows0, nr)], tmp_v.at[pl.ds(0, nr)])
        pltpu.sync_copy(tmp_v.at[pl.ds(0, nr)], acc_sh.at[pl.ds(rows0, nr)])
        pltpu.sync_copy(part_hbm.at[1, pl.ds(rows0, nr)], tmp_v.at[pl.ds(0, nr)])
        pltpu.sync_copy(ramp_hbm.at[pl.ds(rows0, nr)], ramp_v.at[pl.ds(0, nr)])
        pltpu.sync_copy(tmp_v.at[pl.ds(0, nr)],
                        acc_sh.at[ramp_v.at[pl.ds(0, nr)]], add=True)

    @pl.when(s < 15)
    def _():
        build(TROWS)

    @pl.when(s == 15)
    def _():
        build(N_NODES - 15 * TROWS)

    plsc.subcore_barrier()

    iota16 = lax.iota(jnp.int32, 16)
    col0 = jnp.zeros((16,), jnp.int32)
    col1 = col0 + 1
    col2 = col0 + 2
    col3 = col0 + 3

    def chunk(i, carry):
        off = wid * E_PER_W + i * CH
        pltpu.sync_copy(idx_hbm.at[pl.ds(off, CH)], idx_v)
        pltpu.async_copy(acc_sh.at[idx_v], rows_v, sem).wait()

        def group(j, carry2):
            e16 = iota16 + j * 16
            sl = pl.ds(j * 16, 16)
            gx_v[sl] = plsc.load_gather(rows_v, [e16, col0])
            gy_v[sl] = plsc.load_gather(rows_v, [e16, col1])
            gz_v[sl] = plsc.load_gather(rows_v, [e16, col2])
            gc_v[sl] = plsc.load_gather(rows_v, [e16, col3])
            return carry2

        lax.fori_loop(0, G16, group, 0)
        pltpu.sync_copy(gx_v, out_hbm.at[0, pl.ds(off, CH)])
        pltpu.sync_copy(gy_v, out_hbm.at[1, pl.ds(off, CH)])
        pltpu.sync_copy(gz_v, out_hbm.at[2, pl.ds(off, CH)])
        pltpu.sync_copy(gc_v, out_hbm.at[3, pl.ds(off, CH)])
        return carry

    lax.fori_loop(0, N_CH, chunk, 0)


def _final_body(g_ref, wt_ref, b_ref, gam_ref, bet_ref, out_ref):
    g = g_ref[...]
    wt = wt_ref[...]
    w0, w1, w2, w3 = wt[:, 0:1], wt[:, 1:2], wt[:, 2:3], wt[:, 3:4]
    w4, w5, w6, w7 = wt[:, 4:5], wt[:, 5:6], wt[:, 6:7], wt[:, 7:8]
    mx = (w0 - w1 + w2 + w3 - w4 - w5 + w6 - w7) * _INV_SQRT3
    my = (w0 + w1 - w2 + w3 - w4 + w5 - w6 - w7) * _INV_SQRT3
    mz = (w0 + w1 + w2 - w3 + w4 - w5 - w6 - w7) * _INV_SQRT3
    m4 = jnp.concatenate([mx, my, mz, b_ref[...]], axis=1)

    inv = 1.0 / jnp.maximum(g[3:4], 1.0)
    mean4 = jnp.concatenate(
        [g[0:1] * inv, g[1:2] * inv, g[2:3] * inv,
         jnp.ones((1, BT), jnp.float32)], axis=0)
    feat = jnp.dot(m4, mean4, preferred_element_type=jnp.float32)

    o64 = jnp.full((1, C_OUT), 1.0 / C_OUT, jnp.float32)
    ones_col = jnp.ones((C_OUT, 1), jnp.float32)
    mu = jnp.dot(o64, feat, preferred_element_type=jnp.float32)
    s2 = jnp.dot(o64, feat * feat, preferred_element_type=jnp.float32)
    var = s2 - mu * mu
    rs = lax.rsqrt(var + 1e-5)
    mu_b = jnp.dot(ones_col, mu, preferred_element_type=jnp.float32)
    gr = jnp.dot(gam_ref[...], rs, preferred_element_type=jnp.float32)
    bet_b = jnp.dot(bet_ref[...], jnp.ones((1, BT), jnp.float32),
                    preferred_element_type=jnp.float32)
    out_ref[...] = (feat - mu_b) * gr + bet_b


def _final(g_t, Wt, b, gamma, beta):
    return pl.pallas_call(
        _final_body,
        grid=(GRID_E,),
        in_specs=[
            pl.BlockSpec((4, BT), lambda i: (0, i)),
            pl.BlockSpec((C_OUT, 8), lambda i: (0, 0)),
            pl.BlockSpec((C_OUT, 1), lambda i: (0, 0)),
            pl.BlockSpec((C_OUT, 1), lambda i: (0, 0)),
            pl.BlockSpec((C_OUT, 1), lambda i: (0, 0)),
        ],
        out_specs=pl.BlockSpec((C_OUT, BT), lambda i: (0, i)),
        out_shape=jax.ShapeDtypeStruct((C_OUT, E_TOTAL), jnp.float32),
    )(g_t, Wt, b.reshape(C_OUT, 1), gamma.reshape(C_OUT, 1),
      beta.reshape(C_OUT, 1))


def kernel(relative_pos, edge_index_i, W, b, gamma, beta):
    rp_t = relative_pos.T
    dirs_t = _dirs(rp_t)
    zeros = jnp.zeros((N_NODES, 8), jnp.float32)
    parts = _scatter_k(dirs_t, edge_index_i, zeros)
    ramp = jnp.arange(N_NODES, dtype=jnp.int32)
    g_t = _gather_k(parts, edge_index_i, ramp)
    out_t = _final(g_t, W.T, b, gamma, beta)
    return out_t.T


# speedup vs baseline: 15.9337x; 2.3896x over previous
import functools

import jax
import jax.numpy as jnp
from jax import lax
from jax.experimental import pallas as pl
from jax.experimental.pallas import tpu as pltpu
from jax.experimental.pallas import tpu_sc as plsc

N_NODES = 50000
E_TOTAL = 1600000
C_OUT = 64

NW = 32
E_PER_W = E_TOTAL // NW
CH = 2000
N_CH = E_PER_W // CH
G16 = CH // 16

TROWS = 3200

BT = 8192
GRID_E = -(-E_TOTAL // BT)

_INV_SQRT3 = 0.5773502691896258
_RSQRT_SEED = 0x5F3759DF

_sc_mesh = plsc.VectorSubcoreMesh(core_axis_name="c", subcore_axis_name="s")
_sc_params = pltpu.CompilerParams(use_tc_tiling_on_sc=False,
                                  needs_layout_passes=False)


@functools.partial(
    pl.kernel,
    out_type=jax.ShapeDtypeStruct((2, N_NODES, 8), jnp.float32),
    mesh=_sc_mesh,
    compiler_params=_sc_params,
    scratch_types=[
        pltpu.VMEM((CH,), jnp.int32),
        pltpu.VMEM((CH,), jnp.float32),
        pltpu.VMEM((CH,), jnp.float32),
        pltpu.VMEM((CH,), jnp.float32),
        pltpu.VMEM((CH, 8), jnp.float32),
        pltpu.VMEM_SHARED((N_NODES, 8), jnp.float32),
    ],
)
def _scatter_k(rpx_hbm, rpy_hbm, rpz_hbm, idx_hbm, zeros_hbm, part_hbm,
               idx_v, cx_v, cy_v, cz_v, rows_v, acc_sh):
    c = lax.axis_index("c")
    s = lax.axis_index("s")
    wid = s * 2 + c

    @pl.when(s == 0)
    def _():
        pltpu.sync_copy(zeros_hbm, acc_sh)

    iota16 = lax.iota(jnp.int32, 16)
    col0 = jnp.zeros((16,), jnp.int32)
    col1 = col0 + 1
    col2 = col0 + 2
    col3 = col0 + 3
    ones16 = jnp.ones((16,), jnp.float32)

    plsc.subcore_barrier()

    def chunk(i, carry):
        off = wid * E_PER_W + i * CH
        pltpu.sync_copy(idx_hbm.at[pl.ds(off, CH)], idx_v)
        pltpu.sync_copy(rpx_hbm.at[pl.ds(off, CH)], cx_v)
        pltpu.sync_copy(rpy_hbm.at[pl.ds(off, CH)], cy_v)
        pltpu.sync_copy(rpz_hbm.at[pl.ds(off, CH)], cz_v)

        def group(j, carry2):
            sl = pl.ds(j * 16, 16)
            x = cx_v[sl]
            y = cy_v[sl]
            z = cz_v[sl]
            n2 = x * x + y * y + z * z
            seed = _RSQRT_SEED - lax.shift_right_logical(
                plsc.bitcast(n2, jnp.int32), 1)
            r = plsc.bitcast(seed, jnp.float32)
            h = 0.5 * n2
            r = r * (1.5 - h * r * r)
            r = r * (1.5 - h * r * r)
            r = r * (1.5 - h * r * r)
            r = r - 1e-8 * (r * r)
            e16 = iota16 + j * 16
            plsc.store_scatter(rows_v, [e16, col0], x * r)
            plsc.store_scatter(rows_v, [e16, col1], y * r)
            plsc.store_scatter(rows_v, [e16, col2], z * r)
            plsc.store_scatter(rows_v, [e16, col3], ones16)
            return carry2

        lax.fori_loop(0, G16, group, 0)
        pltpu.sync_copy(rows_v, acc_sh.at[idx_v], add=True)
        return carry

    lax.fori_loop(0, N_CH, chunk, 0)
    plsc.subcore_barrier()

    @pl.when(s == 0)
    def _():
        pltpu.sync_copy(acc_sh, part_hbm.at[c])


@functools.partial(
    pl.kernel,
    out_type=[jax.ShapeDtypeStruct((E_TOTAL,), jnp.float32)] * 4,
    mesh=_sc_mesh,
    compiler_params=_sc_params,
    scratch_types=[
        pltpu.VMEM((CH,), jnp.int32),
        pltpu.VMEM((CH, 8), jnp.float32),
        pltpu.VMEM((CH,), jnp.float32),
        pltpu.VMEM((CH,), jnp.float32),
        pltpu.VMEM((CH,), jnp.float32),
        pltpu.VMEM((CH,), jnp.float32),
        pltpu.VMEM((TROWS, 8), jnp.float32),
        pltpu.VMEM((TROWS,), jnp.int32),
        pltpu.VMEM_SHARED((N_NODES, 8), jnp.float32),
        pltpu.SemaphoreType.DMA,
    ],
)
def _gather_k(part_hbm, idx_hbm, ramp_hbm, ox_hbm, oy_hbm, oz_hbm, oc_hbm,
              idx_v, rows_v, gx_v, gy_v, gz_v, gc_v, tmp_v, ramp_v,
              acc_sh, sem):
    c = lax.axis_index("c")
    s = lax.axis_index("s")
    wid = s * 2 + c

    rows0 = s * TROWS

    def build(nr):
        pltpu.sync_copy(part_hbm.at[0, pl.ds(rows0, nr)], tmp_v.at[pl.ds(0, nr)])
        pltpu.sync_copy(tmp_v.at[pl.ds(0, nr)], acc_sh.at[pl.ds(rows0, nr)])
        pltpu.sync_copy(part_hbm.at[1, pl.ds(rows0, nr)], tmp_v.at[pl.ds(0, nr)])
        pltpu.sync_copy(ramp_hbm.at[pl.ds(rows0, nr)], ramp_v.at[pl.ds(0, nr)])
        pltpu.sync_copy(tmp_v.at[pl.ds(0, nr)],
                        acc_sh.at[ramp_v.at[pl.ds(0, nr)]], add=True)

    @pl.when(s < 15)
    def _():
        build(TROWS)

    @pl.when(s == 15)
    def _():
        build(N_NODES - 15 * TROWS)

    plsc.subcore_barrier()

    iota16 = lax.iota(jnp.int32, 16)
    col0 = jnp.zeros((16,), jnp.int32)
    col1 = col0 + 1
    col2 = col0 + 2
    col3 = col0 + 3

    def chunk(i, carry):
        off = wid * E_PER_W + i * CH
        pltpu.sync_copy(idx_hbm.at[pl.ds(off, CH)], idx_v)
        pltpu.async_copy(acc_sh.at[idx_v], rows_v, sem).wait()

        def group(j, carry2):
            e16 = iota16 + j * 16
            sl = pl.ds(j * 16, 16)
            gx_v[sl] = plsc.load_gather(rows_v, [e16, col0])
            gy_v[sl] = plsc.load_gather(rows_v, [e16, col1])
            gz_v[sl] = plsc.load_gather(rows_v, [e16, col2])
            gc_v[sl] = plsc.load_gather(rows_v, [e16, col3])
            return carry2

        lax.fori_loop(0, G16, group, 0)
        pltpu.sync_copy(gx_v, ox_hbm.at[pl.ds(off, CH)])
        pltpu.sync_copy(gy_v, oy_hbm.at[pl.ds(off, CH)])
        pltpu.sync_copy(gz_v, oz_hbm.at[pl.ds(off, CH)])
        pltpu.sync_copy(gc_v, oc_hbm.at[pl.ds(off, CH)])
        return carry

    lax.fori_loop(0, N_CH, chunk, 0)


def _final_body(gx_ref, gy_ref, gz_ref, gc_ref, wt_ref, b_ref, gam_ref,
                bet_ref, out_ref):
    gx = gx_ref[...].reshape(1, BT)
    gy = gy_ref[...].reshape(1, BT)
    gz = gz_ref[...].reshape(1, BT)
    gc = gc_ref[...].reshape(1, BT)
    wt = wt_ref[...]
    w0, w1, w2, w3 = wt[:, 0:1], wt[:, 1:2], wt[:, 2:3], wt[:, 3:4]
    w4, w5, w6, w7 = wt[:, 4:5], wt[:, 5:6], wt[:, 6:7], wt[:, 7:8]
    mx = (w0 - w1 + w2 + w3 - w4 - w5 + w6 - w7) * _INV_SQRT3
    my = (w0 + w1 - w2 + w3 - w4 + w5 - w6 - w7) * _INV_SQRT3
    mz = (w0 + w1 + w2 - w3 + w4 - w5 - w6 - w7) * _INV_SQRT3
    m4 = jnp.concatenate([mx, my, mz, b_ref[...]], axis=1)

    inv = 1.0 / jnp.maximum(gc, 1.0)
    mean4 = jnp.concatenate(
        [gx * inv, gy * inv, gz * inv,
         jnp.ones((1, BT), jnp.float32)], axis=0)
    feat = jnp.dot(m4, mean4, preferred_element_type=jnp.float32)

    o64 = jnp.full((1, C_OUT), 1.0 / C_OUT, jnp.float32)
    ones_col = jnp.ones((C_OUT, 1), jnp.float32)
    mu = jnp.dot(o64, feat, preferred_element_type=jnp.float32)
    s2 = jnp.dot(o64, feat * feat, preferred_element_type=jnp.float32)
    var = s2 - mu * mu
    rs = lax.rsqrt(var + 1e-5)
    mu_b = jnp.dot(ones_col, mu, preferred_element_type=jnp.float32)
    gr = jnp.dot(gam_ref[...], rs, preferred_element_type=jnp.float32)
    bet_b = jnp.dot(bet_ref[...], jnp.ones((1, BT), jnp.float32),
                    preferred_element_type=jnp.float32)
    out_ref[...] = (feat - mu_b) * gr + bet_b


def _final(gx, gy, gz, gc, Wt, b, gamma, beta):
    vec = pl.BlockSpec((BT,), lambda i: (i,))
    return pl.pallas_call(
        _final_body,
        grid=(GRID_E,),
        in_specs=[
            vec, vec, vec, vec,
            pl.BlockSpec((C_OUT, 8), lambda i: (0, 0)),
            pl.BlockSpec((C_OUT, 1), lambda i: (0, 0)),
            pl.BlockSpec((C_OUT, 1), lambda i: (0, 0)),
            pl.BlockSpec((C_OUT, 1), lambda i: (0, 0)),
        ],
        out_specs=pl.BlockSpec((C_OUT, BT), lambda i: (0, i)),
        out_shape=jax.ShapeDtypeStruct((C_OUT, E_TOTAL), jnp.float32),
    )(gx, gy, gz, gc, Wt, b.reshape(C_OUT, 1), gamma.reshape(C_OUT, 1),
      beta.reshape(C_OUT, 1))


def kernel(relative_pos, edge_index_i, W, b, gamma, beta):
    rpx = relative_pos[:, 0]
    rpy = relative_pos[:, 1]
    rpz = relative_pos[:, 2]
    zeros = jnp.zeros((N_NODES, 8), jnp.float32)
    parts = _scatter_k(rpx, rpy, rpz, edge_index_i, zeros)
    ramp = jnp.arange(N_NODES, dtype=jnp.int32)
    gx, gy, gz, gc = _gather_k(parts, edge_index_i, ramp)
    out_t = _final(gx, gy, gz, gc, W.T, b, gamma, beta)
    return out_t.T


# speedup vs baseline: 17.7735x; 1.1155x over previous
import functools

import jax
import jax.numpy as jnp
from jax import lax
from jax.experimental import pallas as pl
from jax.experimental.pallas import tpu as pltpu
from jax.experimental.pallas import tpu_sc as plsc

N_NODES = 50000
E_TOTAL = 1600000
C_OUT = 64

NW = 32
E_PER_W = E_TOTAL // NW
CH = 2000
N_CH = E_PER_W // CH
G16 = CH // 16

TROWS = 3200

BT = 8192
GRID_E = -(-E_TOTAL // BT)

_INV_SQRT3 = 0.5773502691896258
_RSQRT_SEED = 0x5F3759DF

_sc_mesh = plsc.VectorSubcoreMesh(core_axis_name="c", subcore_axis_name="s")
_sc_params = pltpu.CompilerParams(use_tc_tiling_on_sc=False,
                                  needs_layout_passes=False)


@functools.partial(
    pl.kernel,
    out_type=jax.ShapeDtypeStruct((2, N_NODES, 8), jnp.float32),
    mesh=_sc_mesh,
    compiler_params=_sc_params,
    scratch_types=[
        pltpu.VMEM((CH,), jnp.int32),
        pltpu.VMEM((CH,), jnp.float32),
        pltpu.VMEM((CH,), jnp.float32),
        pltpu.VMEM((CH,), jnp.float32),
        pltpu.VMEM((CH, 8), jnp.float32),
        pltpu.VMEM_SHARED((N_NODES, 8), jnp.float32),
    ],
)
def _scatter_k(rpx_hbm, rpy_hbm, rpz_hbm, idx_hbm, zeros_hbm, part_hbm,
               idx_v, cx_v, cy_v, cz_v, rows_v, acc_sh):
    c = lax.axis_index("c")
    s = lax.axis_index("s")
    wid = s * 2 + c

    @pl.when(s == 0)
    def _():
        pltpu.sync_copy(zeros_hbm, acc_sh)

    iota16 = lax.iota(jnp.int32, 16)
    col0 = jnp.zeros((16,), jnp.int32)
    col1 = col0 + 1
    col2 = col0 + 2
    col3 = col0 + 3
    ones16 = jnp.ones((16,), jnp.float32)

    plsc.subcore_barrier()

    def chunk(i, carry):
        off = wid * E_PER_W + i * CH
        pltpu.sync_copy(idx_hbm.at[pl.ds(off, CH)], idx_v)
        pltpu.sync_copy(rpx_hbm.at[pl.ds(off, CH)], cx_v)
        pltpu.sync_copy(rpy_hbm.at[pl.ds(off, CH)], cy_v)
        pltpu.sync_copy(rpz_hbm.at[pl.ds(off, CH)], cz_v)

        def group(j, carry2):
            sl = pl.ds(j * 16, 16)
            x = cx_v[sl]
            y = cy_v[sl]
            z = cz_v[sl]
            n2 = x * x + y * y + z * z
            seed = _RSQRT_SEED - lax.shift_right_logical(
                plsc.bitcast(n2, jnp.int32), 1)
            r = plsc.bitcast(seed, jnp.float32)
            h = 0.5 * n2
            r = r * (1.5 - h * r * r)
            r = r * (1.5 - h * r * r)
            r = r * (1.5 - h * r * r)
            r = r - 1e-8 * (r * r)
            e16 = iota16 + j * 16
            plsc.store_scatter(rows_v, [e16, col0], x * r)
            plsc.store_scatter(rows_v, [e16, col1], y * r)
            plsc.store_scatter(rows_v, [e16, col2], z * r)
            plsc.store_scatter(rows_v, [e16, col3], ones16)
            return carry2

        lax.fori_loop(0, G16, group, 0)
        pltpu.sync_copy(rows_v, acc_sh.at[idx_v], add=True)
        return carry

    lax.fori_loop(0, N_CH, chunk, 0)
    plsc.subcore_barrier()

    @pl.when(s == 0)
    def _():
        pltpu.sync_copy(acc_sh, part_hbm.at[c])


@functools.partial(
    pl.kernel,
    out_type=[jax.ShapeDtypeStruct((E_TOTAL,), jnp.float32)] * 4,
    mesh=_sc_mesh,
    compiler_params=_sc_params,
    scratch_types=[
        pltpu.VMEM((CH,), jnp.int32),
        pltpu.VMEM((CH, 8), jnp.float32),
        pltpu.VMEM((CH,), jnp.float32),
        pltpu.VMEM((CH,), jnp.float32),
        pltpu.VMEM((CH,), jnp.float32),
        pltpu.VMEM((CH,), jnp.float32),
        pltpu.VMEM((TROWS, 8), jnp.float32),
        pltpu.VMEM((TROWS,), jnp.int32),
        pltpu.VMEM_SHARED((N_NODES, 8), jnp.float32),
        pltpu.SemaphoreType.DMA,
    ],
)
def _gather_k(part_hbm, idx_hbm, ramp_hbm, ox_hbm, oy_hbm, oz_hbm, oc_hbm,
              idx_v, rows_v, gx_v, gy_v, gz_v, gc_v, tmp_v, ramp_v,
              acc_sh, sem):
    c = lax.axis_index("c")
    s = lax.axis_index("s")
    wid = s * 2 + c

    rows0 = s * TROWS

    def build(nr):
        pltpu.sync_copy(part_hbm.at[0, pl.ds(rows0, nr)], tmp_v.at[pl.ds(0, nr)])
        pltpu.sync_copy(tmp_v.at[pl.ds(0, nr)], acc_sh.at[pl.ds(rows0, nr)])
        pltpu.sync_copy(part_hbm.at[1, pl.ds(rows0, nr)], tmp_v.at[pl.ds(0, nr)])
        pltpu.sync_copy(ramp_hbm.at[pl.ds(rows0, nr)], ramp_v.at[pl.ds(0, nr)])
        pltpu.sync_copy(tmp_v.at[pl.ds(0, nr)],
                        acc_sh.at[ramp_v.at[pl.ds(0, nr)]], add=True)

    @pl.when(s < 15)
    def _():
        build(TROWS)

    @pl.when(s == 15)
    def _():
        build(N_NODES - 15 * TROWS)

    plsc.subcore_barrier()

    iota16 = lax.iota(jnp.int32, 16)
    col0 = jnp.zeros((16,), jnp.int32)
    col1 = col0 + 1
    col2 = col0 + 2
    col3 = col0 + 3

    def chunk(i, carry):
        off = wid * E_PER_W + i * CH
        pltpu.sync_copy(idx_hbm.at[pl.ds(off, CH)], idx_v)
        pltpu.async_copy(acc_sh.at[idx_v], rows_v, sem).wait()

        def group(j, carry2):
            e16 = iota16 + j * 16
            sl = pl.ds(j * 16, 16)
            gx_v[sl] = plsc.load_gather(rows_v, [e16, col0])
            gy_v[sl] = plsc.load_gather(rows_v, [e16, col1])
            gz_v[sl] = plsc.load_gather(rows_v, [e16, col2])
            gc_v[sl] = plsc.load_gather(rows_v, [e16, col3])
            return carry2

        lax.fori_loop(0, G16, group, 0)
        pltpu.sync_copy(gx_v, ox_hbm.at[pl.ds(off, CH)])
        pltpu.sync_copy(gy_v, oy_hbm.at[pl.ds(off, CH)])
        pltpu.sync_copy(gz_v, oz_hbm.at[pl.ds(off, CH)])
        pltpu.sync_copy(gc_v, oc_hbm.at[pl.ds(off, CH)])
        return carry

    lax.fori_loop(0, N_CH, chunk, 0)


def _final_body(gx_ref, gy_ref, gz_ref, gc_ref, wt_ref, b_ref, gam_ref,
                bet_ref, out_ref):
    f32 = jnp.float32
    gx = gx_ref[...].reshape(1, BT)
    gy = gy_ref[...].reshape(1, BT)
    gz = gz_ref[...].reshape(1, BT)
    gc = gc_ref[...].reshape(1, BT)
    wt = wt_ref[...]
    w0, w1, w2, w3 = wt[:, 0:1], wt[:, 1:2], wt[:, 2:3], wt[:, 3:4]
    w4, w5, w6, w7 = wt[:, 4:5], wt[:, 5:6], wt[:, 6:7], wt[:, 7:8]
    mx = (w0 - w1 + w2 + w3 - w4 - w5 + w6 - w7) * _INV_SQRT3
    my = (w0 + w1 - w2 + w3 - w4 + w5 - w6 - w7) * _INV_SQRT3
    mz = (w0 + w1 + w2 - w3 + w4 - w5 - w6 - w7) * _INV_SQRT3
    m3 = jnp.concatenate([mx, my, mz], axis=1)
    m4 = jnp.concatenate([m3, b_ref[...]], axis=1)

    inv = 1.0 / jnp.maximum(gc, 1.0)
    e1, e2, e3 = gx * inv, gy * inv, gz * inv
    ones_row = jnp.ones((1, BT), f32)
    mean3 = jnp.concatenate([e1, e2, e3], axis=0)
    mean4 = jnp.concatenate([mean3, ones_row], axis=0)

    o64 = jnp.full((1, C_OUT), 1.0 / C_OUT, f32)
    q1 = jnp.dot(o64, m3, preferred_element_type=f32)
    mb = jnp.dot(o64, b_ref[...], preferred_element_type=f32)
    mu = jnp.dot(q1, mean3, preferred_element_type=f32) + mb
    q4 = lax.dot_general(m4, m4, (((0,), (0,)), ((), ())),
                         preferred_element_type=f32) * (1.0 / C_OUT)
    t4 = jnp.dot(q4, mean4, preferred_element_type=f32) * mean4
    s2 = t4[0:1] + t4[1:2] + t4[2:3] + t4[3:4]
    var = s2 - mu * mu
    rs = lax.rsqrt(var + 1e-5)

    gam = gam_ref[...]
    lhs = jnp.concatenate(
        [m3 * gam, b_ref[...] * gam, gam, bet_ref[...]], axis=1)
    rhs = jnp.concatenate(
        [e1 * rs, e2 * rs, e3 * rs, rs, -(mu * rs), ones_row], axis=0)
    out_ref[...] = jnp.dot(lhs, rhs, preferred_element_type=f32)


def _final(gx, gy, gz, gc, Wt, b, gamma, beta):
    vec = pl.BlockSpec((BT,), lambda i: (i,))
    return pl.pallas_call(
        _final_body,
        grid=(GRID_E,),
        in_specs=[
            vec, vec, vec, vec,
            pl.BlockSpec((C_OUT, 8), lambda i: (0, 0)),
            pl.BlockSpec((C_OUT, 1), lambda i: (0, 0)),
            pl.BlockSpec((C_OUT, 1), lambda i: (0, 0)),
            pl.BlockSpec((C_OUT, 1), lambda i: (0, 0)),
        ],
        out_specs=pl.BlockSpec((C_OUT, BT), lambda i: (0, i)),
        out_shape=jax.ShapeDtypeStruct((C_OUT, E_TOTAL), jnp.float32),
    )(gx, gy, gz, gc, Wt, b.reshape(C_OUT, 1), gamma.reshape(C_OUT, 1),
      beta.reshape(C_OUT, 1))


def kernel(relative_pos, edge_index_i, W, b, gamma, beta):
    rpx = relative_pos[:, 0]
    rpy = relative_pos[:, 1]
    rpz = relative_pos[:, 2]
    zeros = jnp.zeros((N_NODES, 8), jnp.float32)
    parts = _scatter_k(rpx, rpy, rpz, edge_index_i, zeros)
    ramp = jnp.arange(N_NODES, dtype=jnp.int32)
    gx, gy, gz, gc = _gather_k(parts, edge_index_i, ramp)
    out_t = _final(gx, gy, gz, gc, W.T, b, gamma, beta)
    return out_t.T


# speedup vs baseline: 21.3703x; 1.2024x over previous
import functools

import jax
import jax.numpy as jnp
from jax import lax
from jax.experimental import pallas as pl
from jax.experimental.pallas import tpu as pltpu
from jax.experimental.pallas import tpu_sc as plsc

N_NODES = 50000
E_TOTAL = 1600000
C_OUT = 64

NW = 32
E_PER_W = E_TOTAL // NW
CH = 2000
N_CH = E_PER_W // CH
G16 = CH // 16

TROWS = 3200

BT = 8192
GRID_E = -(-E_TOTAL // BT)

_INV_SQRT3 = 0.5773502691896258
_RSQRT_SEED = 0x5F3759DF

_sc_mesh = plsc.VectorSubcoreMesh(core_axis_name="c", subcore_axis_name="s")
_sc_params = pltpu.CompilerParams(use_tc_tiling_on_sc=False,
                                  needs_layout_passes=False)


@functools.partial(
    pl.kernel,
    out_type=jax.ShapeDtypeStruct((2, N_NODES, 8), jnp.float32),
    mesh=_sc_mesh,
    compiler_params=_sc_params,
    scratch_types=[
        [pltpu.VMEM((CH,), jnp.int32)] * 2,
        [pltpu.VMEM((CH,), jnp.float32)] * 2,
        [pltpu.VMEM((CH,), jnp.float32)] * 2,
        [pltpu.VMEM((CH,), jnp.float32)] * 2,
        [pltpu.VMEM((CH, 8), jnp.float32)] * 2,
        pltpu.VMEM_SHARED((N_NODES, 8), jnp.float32),
        [pltpu.SemaphoreType.DMA] * 2,
        [pltpu.SemaphoreType.DMA] * 2,
    ],
)
def _scatter_k(rpx_hbm, rpy_hbm, rpz_hbm, idx_hbm, zeros_hbm, part_hbm,
               idx_v, cx_v, cy_v, cz_v, rows_v, acc_sh, sem_i, sem_o):
    c = lax.axis_index("c")
    s = lax.axis_index("s")
    wid = s * 2 + c

    @pl.when(s == 0)
    def _():
        pltpu.sync_copy(zeros_hbm, acc_sh)

    iota16 = lax.iota(jnp.int32, 16)
    col0 = jnp.zeros((16,), jnp.int32)
    col1 = col0 + 1
    col2 = col0 + 2
    col3 = col0 + 3
    ones16 = jnp.ones((16,), jnp.float32)

    plsc.subcore_barrier()

    def start_in(ci, p):
        off = wid * E_PER_W + ci * CH
        pltpu.async_copy(idx_hbm.at[pl.ds(off, CH)], idx_v[p], sem_i[p])
        pltpu.async_copy(rpx_hbm.at[pl.ds(off, CH)], cx_v[p], sem_i[p])
        pltpu.async_copy(rpy_hbm.at[pl.ds(off, CH)], cy_v[p], sem_i[p])
        pltpu.async_copy(rpz_hbm.at[pl.ds(off, CH)], cz_v[p], sem_i[p])

    def wait_in(p):
        sl = pl.ds(0, CH)
        pltpu.make_async_copy(idx_hbm.at[sl], idx_v[p], sem_i[p]).wait()
        pltpu.make_async_copy(rpx_hbm.at[sl], cx_v[p], sem_i[p]).wait()
        pltpu.make_async_copy(rpy_hbm.at[sl], cy_v[p], sem_i[p]).wait()
        pltpu.make_async_copy(rpz_hbm.at[sl], cz_v[p], sem_i[p]).wait()

    def start_out(p):
        pltpu.async_copy(rows_v[p], acc_sh.at[idx_v[p]], sem_o[p], add=True)

    def wait_out(p):
        pltpu.make_async_copy(rows_v[p], acc_sh.at[idx_v[p]], sem_o[p]).wait()

    def interleave(p):
        def group(j, carry2):
            sl = pl.ds(j * 16, 16)
            x = cx_v[p][sl]
            y = cy_v[p][sl]
            z = cz_v[p][sl]
            n2 = x * x + y * y + z * z
            seed = _RSQRT_SEED - lax.shift_right_logical(
                plsc.bitcast(n2, jnp.int32), 1)
            r = plsc.bitcast(seed, jnp.float32)
            h = 0.5 * n2
            r = r * (1.5 - h * r * r)
            r = r * (1.5 - h * r * r)
            r = r - 1e-8 * (r * r)
            e16 = iota16 + j * 16
            plsc.store_scatter(rows_v[p], [e16, col0], x * r)
            plsc.store_scatter(rows_v[p], [e16, col1], y * r)
            plsc.store_scatter(rows_v[p], [e16, col2], z * r)
            plsc.store_scatter(rows_v[p], [e16, col3], ones16)
            return carry2

        lax.fori_loop(0, G16, group, 0)

    start_in(0, 0)

    def body(ii, carry):
        c0 = 2 * ii

        @pl.when(ii > 0)
        def _():
            wait_out(1)

        wait_in(0)
        start_in(c0 + 1, 1)
        interleave(0)
        start_out(0)
        wait_in(1)
        interleave(1)
        wait_out(0)

        @pl.when(c0 + 2 < N_CH)
        def _():
            start_in(c0 + 2, 0)

        start_out(1)
        return carry

    lax.fori_loop(0, (N_CH - 1) // 2, body, 0)
    wait_out(1)
    wait_in(0)
    interleave(0)
    start_out(0)
    wait_out(0)

    plsc.subcore_barrier()

    @pl.when(s == 0)
    def _():
        pltpu.sync_copy(acc_sh, part_hbm.at[c])


@functools.partial(
    pl.kernel,
    out_type=[jax.ShapeDtypeStruct((E_TOTAL,), jnp.float32)] * 4,
    mesh=_sc_mesh,
    compiler_params=_sc_params,
    scratch_types=[
        [pltpu.VMEM((CH,), jnp.int32)] * 2,
        [pltpu.VMEM((CH, 8), jnp.float32)] * 2,
        [pltpu.VMEM((CH,), jnp.float32)] * 2,
        [pltpu.VMEM((CH,), jnp.float32)] * 2,
        [pltpu.VMEM((CH,), jnp.float32)] * 2,
        [pltpu.VMEM((CH,), jnp.float32)] * 2,
        pltpu.VMEM((TROWS, 8), jnp.float32),
        pltpu.VMEM((TROWS,), jnp.int32),
        pltpu.VMEM_SHARED((N_NODES, 8), jnp.float32),
        [pltpu.SemaphoreType.DMA] * 2,
        [pltpu.SemaphoreType.DMA] * 2,
    ],
)
def _gather_k(part_hbm, idx_hbm, ramp_hbm, ox_hbm, oy_hbm, oz_hbm, oc_hbm,
              idx_v, rows_v, gx_v, gy_v, gz_v, gc_v, tmp_v, ramp_v,
              acc_sh, sem_g, sem_o):
    c = lax.axis_index("c")
    s = lax.axis_index("s")
    wid = s * 2 + c

    rows0 = s * TROWS

    def build(nr):
        pltpu.sync_copy(part_hbm.at[0, pl.ds(rows0, nr)], tmp_v.at[pl.ds(0, nr)])
        pltpu.sync_copy(tmp_v.at[pl.ds(0, nr)], acc_sh.at[pl.ds(rows0, nr)])
        pltpu.sync_copy(part_hbm.at[1, pl.ds(rows0, nr)], tmp_v.at[pl.ds(0, nr)])
        pltpu.sync_copy(ramp_hbm.at[pl.ds(rows0, nr)], ramp_v.at[pl.ds(0, nr)])
        pltpu.sync_copy(tmp_v.at[pl.ds(0, nr)],
                        acc_sh.at[ramp_v.at[pl.ds(0, nr)]], add=True)

    @pl.when(s < 15)
    def _():
        build(TROWS)

    @pl.when(s == 15)
    def _():
        build(N_NODES - 15 * TROWS)

    plsc.subcore_barrier()

    iota16 = lax.iota(jnp.int32, 16)
    col0 = jnp.zeros((16,), jnp.int32)
    col1 = col0 + 1
    col2 = col0 + 2
    col3 = col0 + 3

    def start_gather(ci, p):
        off = wid * E_PER_W + ci * CH
        pltpu.sync_copy(idx_hbm.at[pl.ds(off, CH)], idx_v[p])
        pltpu.async_copy(acc_sh.at[idx_v[p]], rows_v[p], sem_g[p])

    def wait_gather(p):
        pltpu.make_async_copy(acc_sh.at[idx_v[p]], rows_v[p], sem_g[p]).wait()

    def start_out(ci, p):
        off = wid * E_PER_W + ci * CH
        pltpu.async_copy(gx_v[p], ox_hbm.at[pl.ds(off, CH)], sem_o[p])
        pltpu.async_copy(gy_v[p], oy_hbm.at[pl.ds(off, CH)], sem_o[p])
        pltpu.async_copy(gz_v[p], oz_hbm.at[pl.ds(off, CH)], sem_o[p])
        pltpu.async_copy(gc_v[p], oc_hbm.at[pl.ds(off, CH)], sem_o[p])

    def wait_out(p):
        sl = pl.ds(0, CH)
        pltpu.make_async_copy(gx_v[p], ox_hbm.at[sl], sem_o[p]).wait()
        pltpu.make_async_copy(gy_v[p], oy_hbm.at[sl], sem_o[p]).wait()
        pltpu.make_async_copy(gz_v[p], oz_hbm.at[sl], sem_o[p]).wait()
        pltpu.make_async_copy(gc_v[p], oc_hbm.at[sl], sem_o[p]).wait()

    def deint(p):
        def group(j, carry2):
            e16 = iota16 + j * 16
            sl = pl.ds(j * 16, 16)
            gx_v[p][sl] = plsc.load_gather(rows_v[p], [e16, col0])
            gy_v[p][sl] = plsc.load_gather(rows_v[p], [e16, col1])
            gz_v[p][sl] = plsc.load_gather(rows_v[p], [e16, col2])
            gc_v[p][sl] = plsc.load_gather(rows_v[p], [e16, col3])
            return carry2

        lax.fori_loop(0, G16, group, 0)

    start_gather(0, 0)

    def body(ii, carry):
        c0 = 2 * ii
        start_gather(c0 + 1, 1)
        wait_gather(0)

        @pl.when(ii > 0)
        def _():
            wait_out(0)

        deint(0)
        start_out(c0, 0)

        @pl.when(c0 + 2 < N_CH)
        def _():
            start_gather(c0 + 2, 0)

        wait_gather(1)

        @pl.when(ii > 0)
        def _():
            wait_out(1)

        deint(1)
        start_out(c0 + 1, 1)
        return carry

    lax.fori_loop(0, (N_CH - 1) // 2, body, 0)
    wait_gather(0)
    wait_out(0)
    deint(0)
    start_out(N_CH - 1, 0)
    wait_out(0)
    wait_out(1)


def _final_body(gx_ref, gy_ref, gz_ref, gc_ref, wt_ref, b_ref, gam_ref,
                bet_ref, out_ref):
    f32 = jnp.float32
    gx = gx_ref[...].reshape(1, BT)
    gy = gy_ref[...].reshape(1, BT)
    gz = gz_ref[...].reshape(1, BT)
    gc = gc_ref[...].reshape(1, BT)
    wt = wt_ref[...]
    w0, w1, w2, w3 = wt[:, 0:1], wt[:, 1:2], wt[:, 2:3], wt[:, 3:4]
    w4, w5, w6, w7 = wt[:, 4:5], wt[:, 5:6], wt[:, 6:7], wt[:, 7:8]
    mx = (w0 - w1 + w2 + w3 - w4 - w5 + w6 - w7) * _INV_SQRT3
    my = (w0 + w1 - w2 + w3 - w4 + w5 - w6 - w7) * _INV_SQRT3
    mz = (w0 + w1 + w2 - w3 + w4 - w5 - w6 - w7) * _INV_SQRT3
    m3 = jnp.concatenate([mx, my, mz], axis=1)
    m4 = jnp.concatenate([m3, b_ref[...]], axis=1)

    inv = 1.0 / jnp.maximum(gc, 1.0)
    e1, e2, e3 = gx * inv, gy * inv, gz * inv
    ones_row = jnp.ones((1, BT), f32)
    mean3 = jnp.concatenate([e1, e2, e3], axis=0)
    mean4 = jnp.concatenate([mean3, ones_row], axis=0)

    o64 = jnp.full((1, C_OUT), 1.0 / C_OUT, f32)
    q1 = jnp.dot(o64, m3, preferred_element_type=f32)
    mb = jnp.dot(o64, b_ref[...], preferred_element_type=f32)
    mu = jnp.dot(q1, mean3, preferred_element_type=f32) + mb
    q4 = lax.dot_general(m4, m4, (((0,), (0,)), ((), ())),
                         preferred_element_type=f32) * (1.0 / C_OUT)
    t4 = jnp.dot(q4, mean4, preferred_element_type=f32) * mean4
    s2 = t4[0:1] + t4[1:2] + t4[2:3] + t4[3:4]
    var = s2 - mu * mu
    rs = lax.rsqrt(var + 1e-5)

    gam = gam_ref[...]
    lhs = jnp.concatenate(
        [m3 * gam, b_ref[...] * gam, gam, bet_ref[...]], axis=1)
    rhs = jnp.concatenate(
        [e1 * rs, e2 * rs, e3 * rs, rs, -(mu * rs), ones_row], axis=0)
    out_ref[...] = jnp.dot(lhs, rhs, preferred_element_type=f32)


def _final(gx, gy, gz, gc, Wt, b, gamma, beta):
    vec = pl.BlockSpec((BT,), lambda i: (i,))
    return pl.pallas_call(
        _final_body,
        grid=(GRID_E,),
        in_specs=[
            vec, vec, vec, vec,
            pl.BlockSpec((C_OUT, 8), lambda i: (0, 0)),
            pl.BlockSpec((C_OUT, 1), lambda i: (0, 0)),
            pl.BlockSpec((C_OUT, 1), lambda i: (0, 0)),
            pl.BlockSpec((C_OUT, 1), lambda i: (0, 0)),
        ],
        out_specs=pl.BlockSpec((C_OUT, BT), lambda i: (0, i)),
        out_shape=jax.ShapeDtypeStruct((C_OUT, E_TOTAL), jnp.float32),
    )(gx, gy, gz, gc, Wt, b.reshape(C_OUT, 1), gamma.reshape(C_OUT, 1),
      beta.reshape(C_OUT, 1))


def kernel(relative_pos, edge_index_i, W, b, gamma, beta):
    rpx = relative_pos[:, 0]
    rpy = relative_pos[:, 1]
    rpz = relative_pos[:, 2]
    zeros = jnp.zeros((N_NODES, 8), jnp.float32)
    parts = _scatter_k(rpx, rpy, rpz, edge_index_i, zeros)
    ramp = jnp.arange(N_NODES, dtype=jnp.int32)
    gx, gy, gz, gc = _gather_k(parts, edge_index_i, ramp)
    out_t = _final(gx, gy, gz, gc, W.T, b, gamma, beta)
    return out_t.T


# speedup vs baseline: 24.4458x; 1.1439x over previous
import functools

import jax
import jax.numpy as jnp
from jax import lax
from jax.experimental import pallas as pl
from jax.experimental.pallas import tpu as pltpu
from jax.experimental.pallas import tpu_sc as plsc

N_NODES = 50000
E_TOTAL = 1600000
C_OUT = 64

NW = 32
E_PER_W = E_TOTAL // NW
CH = 2000
N_CH = E_PER_W // CH
G16 = CH // 16

TROWS = 3200

BT = 16384
GRID_E = -(-E_TOTAL // BT)

_INV_SQRT3 = 0.5773502691896258
_RSQRT_SEED = 0x5F3759DF

_sc_mesh = plsc.VectorSubcoreMesh(core_axis_name="c", subcore_axis_name="s")
_sc_params = pltpu.CompilerParams(use_tc_tiling_on_sc=False,
                                  needs_layout_passes=False)


@functools.partial(
    pl.kernel,
    out_type=jax.ShapeDtypeStruct((2, N_NODES, 8), jnp.float32),
    mesh=_sc_mesh,
    compiler_params=_sc_params,
    scratch_types=[
        [pltpu.VMEM((CH,), jnp.int32)] * 2,
        [pltpu.VMEM((CH,), jnp.float32)] * 2,
        [pltpu.VMEM((CH,), jnp.float32)] * 2,
        [pltpu.VMEM((CH,), jnp.float32)] * 2,
        [pltpu.VMEM((CH, 8), jnp.float32)] * 2,
        pltpu.VMEM_SHARED((N_NODES, 8), jnp.float32),
        [pltpu.SemaphoreType.DMA] * 2,
        [pltpu.SemaphoreType.DMA] * 2,
    ],
)
def _scatter_k(rpx_hbm, rpy_hbm, rpz_hbm, idx_hbm, zeros_hbm, part_hbm,
               idx_v, cx_v, cy_v, cz_v, rows_v, acc_sh, sem_i, sem_o):
    c = lax.axis_index("c")
    s = lax.axis_index("s")
    wid = s * 2 + c

    @pl.when(s == 0)
    def _():
        pltpu.sync_copy(zeros_hbm, acc_sh)

    iota16 = lax.iota(jnp.int32, 16)
    col0 = jnp.zeros((16,), jnp.int32)
    col1 = col0 + 1
    col2 = col0 + 2
    col3 = col0 + 3
    ones16 = jnp.ones((16,), jnp.float32)

    plsc.subcore_barrier()

    def start_in(ci, p):
        off = wid * E_PER_W + ci * CH
        pltpu.async_copy(idx_hbm.at[pl.ds(off, CH)], idx_v[p], sem_i[p])
        pltpu.async_copy(rpx_hbm.at[pl.ds(off, CH)], cx_v[p], sem_i[p])
        pltpu.async_copy(rpy_hbm.at[pl.ds(off, CH)], cy_v[p], sem_i[p])
        pltpu.async_copy(rpz_hbm.at[pl.ds(off, CH)], cz_v[p], sem_i[p])

    def wait_in(p):
        sl = pl.ds(0, CH)
        pltpu.make_async_copy(idx_hbm.at[sl], idx_v[p], sem_i[p]).wait()
        pltpu.make_async_copy(rpx_hbm.at[sl], cx_v[p], sem_i[p]).wait()
        pltpu.make_async_copy(rpy_hbm.at[sl], cy_v[p], sem_i[p]).wait()
        pltpu.make_async_copy(rpz_hbm.at[sl], cz_v[p], sem_i[p]).wait()

    def start_out(p):
        pltpu.async_copy(rows_v[p], acc_sh.at[idx_v[p]], sem_o[p], add=True)

    def wait_out(p):
        pltpu.make_async_copy(rows_v[p], acc_sh.at[idx_v[p]], sem_o[p]).wait()

    def interleave(p):
        def group(j, carry2):
            sl = pl.ds(j * 16, 16)
            x = cx_v[p][sl]
            y = cy_v[p][sl]
            z = cz_v[p][sl]
            n2 = x * x + y * y + z * z
            seed = _RSQRT_SEED - lax.shift_right_logical(
                plsc.bitcast(n2, jnp.int32), 1)
            r = plsc.bitcast(seed, jnp.float32)
            h = 0.5 * n2
            r = r * (1.5 - h * r * r)
            r = r * (1.5 - h * r * r)
            r = r - 1e-8 * (r * r)
            e16 = iota16 + j * 16
            plsc.store_scatter(rows_v[p], [e16, col0], x * r)
            plsc.store_scatter(rows_v[p], [e16, col1], y * r)
            plsc.store_scatter(rows_v[p], [e16, col2], z * r)
            plsc.store_scatter(rows_v[p], [e16, col3], ones16)
            return carry2

        lax.fori_loop(0, G16, group, 0)

    start_in(0, 0)

    def body(ii, carry):
        c0 = 2 * ii

        @pl.when(ii > 0)
        def _():
            wait_out(1)

        wait_in(0)
        start_in(c0 + 1, 1)
        interleave(0)
        start_out(0)
        wait_in(1)
        interleave(1)
        wait_out(0)

        @pl.when(c0 + 2 < N_CH)
        def _():
            start_in(c0 + 2, 0)

        start_out(1)
        return carry

    lax.fori_loop(0, (N_CH - 1) // 2, body, 0)
    wait_out(1)
    wait_in(0)
    interleave(0)
    start_out(0)
    wait_out(0)

    plsc.subcore_barrier()

    @pl.when(s == 0)
    def _():
        pltpu.sync_copy(acc_sh, part_hbm.at[c])


@functools.partial(
    pl.kernel,
    out_type=[jax.ShapeDtypeStruct((E_TOTAL,), jnp.float32)] * 4,
    mesh=_sc_mesh,
    compiler_params=_sc_params,
    scratch_types=[
        [pltpu.VMEM((CH,), jnp.int32)] * 2,
        [pltpu.VMEM((CH, 8), jnp.float32)] * 2,
        [pltpu.VMEM((CH,), jnp.float32)] * 2,
        [pltpu.VMEM((CH,), jnp.float32)] * 2,
        [pltpu.VMEM((CH,), jnp.float32)] * 2,
        [pltpu.VMEM((CH,), jnp.float32)] * 2,
        pltpu.VMEM((TROWS, 8), jnp.float32),
        pltpu.VMEM((TROWS,), jnp.int32),
        pltpu.VMEM_SHARED((N_NODES, 8), jnp.float32),
        [pltpu.SemaphoreType.DMA] * 2,
        [pltpu.SemaphoreType.DMA] * 2,
    ],
)
def _gather_k(part_hbm, idx_hbm, ramp_hbm, ox_hbm, oy_hbm, oz_hbm, oc_hbm,
              idx_v, rows_v, gx_v, gy_v, gz_v, gc_v, tmp_v, ramp_v,
              acc_sh, sem_g, sem_o):
    c = lax.axis_index("c")
    s = lax.axis_index("s")
    wid = s * 2 + c

    rows0 = s * TROWS

    def build(nr):
        pltpu.sync_copy(part_hbm.at[0, pl.ds(rows0, nr)], tmp_v.at[pl.ds(0, nr)])
        pltpu.sync_copy(tmp_v.at[pl.ds(0, nr)], acc_sh.at[pl.ds(rows0, nr)])
        pltpu.sync_copy(part_hbm.at[1, pl.ds(rows0, nr)], tmp_v.at[pl.ds(0, nr)])
        pltpu.sync_copy(ramp_hbm.at[pl.ds(rows0, nr)], ramp_v.at[pl.ds(0, nr)])
        pltpu.sync_copy(tmp_v.at[pl.ds(0, nr)],
                        acc_sh.at[ramp_v.at[pl.ds(0, nr)]], add=True)

    @pl.when(s < 15)
    def _():
        build(TROWS)

    @pl.when(s == 15)
    def _():
        build(N_NODES - 15 * TROWS)

    plsc.subcore_barrier()

    iota16 = lax.iota(jnp.int32, 16)
    col0 = jnp.zeros((16,), jnp.int32)
    col1 = col0 + 1
    col2 = col0 + 2
    col3 = col0 + 3

    def start_gather(ci, p):
        off = wid * E_PER_W + ci * CH
        pltpu.sync_copy(idx_hbm.at[pl.ds(off, CH)], idx_v[p])
        pltpu.async_copy(acc_sh.at[idx_v[p]], rows_v[p], sem_g[p])

    def wait_gather(p):
        pltpu.make_async_copy(acc_sh.at[idx_v[p]], rows_v[p], sem_g[p]).wait()

    def start_out(ci, p):
        off = wid * E_PER_W + ci * CH
        pltpu.async_copy(gx_v[p], ox_hbm.at[pl.ds(off, CH)], sem_o[p])
        pltpu.async_copy(gy_v[p], oy_hbm.at[pl.ds(off, CH)], sem_o[p])
        pltpu.async_copy(gz_v[p], oz_hbm.at[pl.ds(off, CH)], sem_o[p])
        pltpu.async_copy(gc_v[p], oc_hbm.at[pl.ds(off, CH)], sem_o[p])

    def wait_out(p):
        sl = pl.ds(0, CH)
        pltpu.make_async_copy(gx_v[p], ox_hbm.at[sl], sem_o[p]).wait()
        pltpu.make_async_copy(gy_v[p], oy_hbm.at[sl], sem_o[p]).wait()
        pltpu.make_async_copy(gz_v[p], oz_hbm.at[sl], sem_o[p]).wait()
        pltpu.make_async_copy(gc_v[p], oc_hbm.at[sl], sem_o[p]).wait()

    def deint(p):
        def group(j, carry2):
            e16 = iota16 + j * 16
            sl = pl.ds(j * 16, 16)
            gx_v[p][sl] = plsc.load_gather(rows_v[p], [e16, col0])
            gy_v[p][sl] = plsc.load_gather(rows_v[p], [e16, col1])
            gz_v[p][sl] = plsc.load_gather(rows_v[p], [e16, col2])
            gc_v[p][sl] = plsc.load_gather(rows_v[p], [e16, col3])
            return carry2

        lax.fori_loop(0, G16, group, 0)

    start_gather(0, 0)

    def body(ii, carry):
        c0 = 2 * ii
        start_gather(c0 + 1, 1)
        wait_gather(0)

        @pl.when(ii > 0)
        def _():
            wait_out(0)

        deint(0)
        start_out(c0, 0)

        @pl.when(c0 + 2 < N_CH)
        def _():
            start_gather(c0 + 2, 0)

        wait_gather(1)

        @pl.when(ii > 0)
        def _():
            wait_out(1)

        deint(1)
        start_out(c0 + 1, 1)
        return carry

    lax.fori_loop(0, (N_CH - 1) // 2, body, 0)
    wait_gather(0)
    wait_out(0)
    deint(0)
    start_out(N_CH - 1, 0)
    wait_out(0)
    wait_out(1)


def _final_body(gx_ref, gy_ref, gz_ref, gc_ref, wt_ref, b_ref, gam_ref,
                bet_ref, out_ref):
    f32 = jnp.float32
    gx = gx_ref[...].reshape(1, BT)
    gy = gy_ref[...].reshape(1, BT)
    gz = gz_ref[...].reshape(1, BT)
    gc = gc_ref[...].reshape(1, BT)
    wt = wt_ref[...]
    w0, w1, w2, w3 = wt[:, 0:1], wt[:, 1:2], wt[:, 2:3], wt[:, 3:4]
    w4, w5, w6, w7 = wt[:, 4:5], wt[:, 5:6], wt[:, 6:7], wt[:, 7:8]
    mx = (w0 - w1 + w2 + w3 - w4 - w5 + w6 - w7) * _INV_SQRT3
    my = (w0 + w1 - w2 + w3 - w4 + w5 - w6 - w7) * _INV_SQRT3
    mz = (w0 + w1 + w2 - w3 + w4 - w5 - w6 - w7) * _INV_SQRT3
    m3 = jnp.concatenate([mx, my, mz], axis=1)
    m4 = jnp.concatenate([m3, b_ref[...]], axis=1)

    inv = 1.0 / jnp.maximum(gc, 1.0)
    e1, e2, e3 = gx * inv, gy * inv, gz * inv
    ones_row = jnp.ones((1, BT), f32)
    mean3 = jnp.concatenate([e1, e2, e3], axis=0)
    mean4 = jnp.concatenate([mean3, ones_row], axis=0)

    o64 = jnp.full((1, C_OUT), 1.0 / C_OUT, f32)
    q1 = jnp.dot(o64, m3, preferred_element_type=f32)
    mb = jnp.dot(o64, b_ref[...], preferred_element_type=f32)
    mu = jnp.dot(q1, mean3, preferred_element_type=f32) + mb
    q4 = lax.dot_general(m4, m4, (((0,), (0,)), ((), ())),
                         preferred_element_type=f32) * (1.0 / C_OUT)
    t4 = jnp.dot(q4, mean4, preferred_element_type=f32) * mean4
    s2 = t4[0:1] + t4[1:2] + t4[2:3] + t4[3:4]
    var = s2 - mu * mu
    rs = lax.rsqrt(var + 1e-5)

    gam = gam_ref[...]
    lhs = jnp.concatenate(
        [m3 * gam, b_ref[...] * gam, gam, bet_ref[...]], axis=1)
    rhs = jnp.concatenate(
        [e1 * rs, e2 * rs, e3 * rs, rs, -(mu * rs), ones_row], axis=0)
    out_ref[...] = jnp.dot(lhs, rhs, preferred_element_type=f32)


def _final(gx, gy, gz, gc, Wt, b, gamma, beta):
    vec = pl.BlockSpec((BT,), lambda i: (i,))
    return pl.pallas_call(
        _final_body,
        grid=(GRID_E,),
        in_specs=[
            vec, vec, vec, vec,
            pl.BlockSpec((C_OUT, 8), lambda i: (0, 0)),
            pl.BlockSpec((C_OUT, 1), lambda i: (0, 0)),
            pl.BlockSpec((C_OUT, 1), lambda i: (0, 0)),
            pl.BlockSpec((C_OUT, 1), lambda i: (0, 0)),
        ],
        out_specs=pl.BlockSpec((C_OUT, BT), lambda i: (0, i)),
        out_shape=jax.ShapeDtypeStruct((C_OUT, E_TOTAL), jnp.float32),
    )(gx, gy, gz, gc, Wt, b.reshape(C_OUT, 1), gamma.reshape(C_OUT, 1),
      beta.reshape(C_OUT, 1))


def kernel(relative_pos, edge_index_i, W, b, gamma, beta):
    rpx = relative_pos[:, 0]
    rpy = relative_pos[:, 1]
    rpz = relative_pos[:, 2]
    zeros = jnp.zeros((N_NODES, 8), jnp.float32)
    parts = _scatter_k(rpx, rpy, rpz, edge_index_i, zeros)
    ramp = jnp.arange(N_NODES, dtype=jnp.int32)
    gx, gy, gz, gc = _gather_k(parts, edge_index_i, ramp)
    out_t = _final(gx, gy, gz, gc, W.T, b, gamma, beta)
    return out_t.T
